# Initial kernel scaffold; baseline (speedup 1.0000x reference)
import jax
import jax.numpy as jnp
from jax.experimental import pallas as pl


def kernel(d_node, q_node, d_node_mask, q_node_mask, graph, adapt_W, adapt_b, Wk, bk, Wq, bq, Wv, bv, Wa, ba, rel_pri, rel_att, rel_msg, skip, ln_g, ln_b, out_W, out_b):
    raise NotImplementedError("write your pallas kernel here")


# speedup vs baseline: 3.0108x; 3.0108x over previous
import functools

import jax
import jax.numpy as jnp
import numpy as np
from jax.experimental import pallas as pl

B, D, Q, DM, H = 8, 512, 128, 128, 2
DK = DM // H
SQRT_DK = float(np.sqrt(DK))
EPS = 1e-5
_RELS = [(0, 1, 0), (0, 0, 1), (1, 1, 2), (1, 0, 3),
         (0, 1, 4), (0, 0, 5), (1, 1, 6), (1, 0, 7)]


def _hgt_body(dn_ref, qn_ref, dm_ref, qm_ref, g_ref,
              aW_ref, ab_ref, Wk_ref, bk_ref, Wq_ref, bq_ref, Wv_ref, bv_ref,
              Wa_ref, ba_ref, hmix_ref, A_ref, M_ref,
              ln_g_ref, ln_b_ref, oW_ref, ob_ref,
              outd_ref, outq_ref):
    f32 = jnp.float32

    def mm(a, b):
        return jax.lax.dot_general(a, b, (((1,), (0,)), ((), ())),
                                   preferred_element_type=f32)

    def mm_nt(a, b):
        return jax.lax.dot_general(a, b, (((1,), (1,)), ((), ())),
                                   preferred_element_type=f32)

    def mm_tn(a, b):
        return jax.lax.dot_general(a, b, (((0,), (0,)), ((), ())),
                                   preferred_element_type=f32)

    dn = dn_ref[0]
    qn = qn_ref[0]
    dmf = dm_ref[0].astype(f32)
    qmf = qm_ref[0].astype(f32)
    g = g_ref[0] > 0

    def gelu_exact(x):
        return x * 0.5 * (1.0 + jax.lax.erf(x * np.float32(1.0 / np.sqrt(2.0))))

    h = [gelu_exact(mm(dn, aW_ref[0]) + ab_ref[0][None, :]),
         gelu_exact(mm(qn, aW_ref[1]) + ab_ref[1][None, :])]
    kb = [mm(h[t], Wk_ref[t]) + bk_ref[t][None, :] for t in (0, 1)]
    qb = [mm(h[t], Wq_ref[t]) + bq_ref[t][None, :] for t in (0, 1)]
    vb = [mm(h[t], Wv_ref[t]) + bv_ref[t][None, :] for t in (0, 1)]

    eye_d = (jax.lax.broadcasted_iota(jnp.int32, (D, D), 0)
             == jax.lax.broadcasted_iota(jnp.int32, (D, D), 1))
    eye_q = (jax.lax.broadcasted_iota(jnp.int32, (Q, Q), 0)
             == jax.lax.broadcasted_iota(jnp.int32, (Q, Q), 1))
    pair = {
        (0, 0): (mm_tn(dmf, dmf) > 0) & ~eye_d,
        (1, 1): (mm_tn(qmf, qmf) > 0) & ~eye_q,
        (0, 1): mm_tn(dmf, qmf) > 0,
        (1, 0): mm_tn(qmf, dmf) > 0,
    }
    gblk = {
        (0, 0): g[:D, :D], (0, 1): g[:D, D:],
        (1, 0): g[D:, :D], (1, 1): g[D:, D:],
    }

    agg = [None, None]
    for (s, t, e) in _RELS:
        lg = e < 4
        gm = gblk[(s, t)]
        mask = pair[(s, t)] & (gm if lg else ~gm)
        heads = []
        for hh in range(H):
            sl = slice(hh * DK, (hh + 1) * DK)
            k_sh = mm(kb[s][:, sl], A_ref[e, hh])
            v_sh = mm(vb[s][:, sl], M_ref[e, hh])
            scores = mm_nt(k_sh, qb[t][:, sl])
            scores = jnp.where(mask, scores, -1e30)
            mx = jnp.max(scores, axis=0, keepdims=True)
            p = jnp.where(mask, jnp.exp(scores - mx), 0.0)
            den = jnp.sum(p, axis=0)[:, None]
            aggh = mm_tn(p, v_sh)
            heads.append(aggh / jnp.maximum(den, 1e-30))
        contrib = jnp.concatenate(heads, axis=1)
        agg[t] = contrib if agg[t] is None else agg[t] + contrib

    for t, out_ref in ((0, outd_ref), (1, outq_ref)):
        tt = agg[t] * 0.25
        trans = mm(tt, Wa_ref[t]) + ba_ref[t][None, :] + h[t] * hmix_ref[t][None, :]
        mu = jnp.mean(trans, axis=-1, keepdims=True)
        cent = trans - mu
        var = jnp.mean(cent * cent, axis=-1, keepdims=True)
        nh = cent * jax.lax.rsqrt(var + EPS) * ln_g_ref[t][None, :] + ln_b_ref[t][None, :]
        out_ref[0] = mm(nh, oW_ref[...]) + ob_ref[0][None, :]


@jax.jit
def kernel(d_node, q_node, d_node_mask, q_node_mask, graph,
           adapt_W, adapt_b, Wk, bk, Wq, bq, Wv, bv, Wa, ba,
           rel_pri, rel_att, rel_msg, skip, ln_g, ln_b, out_W, out_b):
    f32 = jnp.float32
    A_eff = rel_att * (rel_pri / SQRT_DK)[:, :, None, None]
    alpha = jax.nn.sigmoid(skip)
    Wa_eff = Wa * alpha[:, None, None]
    ba_eff = ba * alpha[:, None]
    hmix = jnp.broadcast_to((1.0 - alpha)[:, None], (2, DM)).astype(f32)
    dmask3 = d_node_mask.reshape(B, 1, D)
    qmask3 = q_node_mask.reshape(B, 1, Q)
    ob2 = out_b.reshape(1, DM)

    def bspec(shape, batched):
        if batched:
            return pl.BlockSpec((1,) + shape[1:],
                                lambda b: (b,) + (0,) * (len(shape) - 1))
        return pl.BlockSpec(shape, lambda b: (0,) * len(shape))

    in_specs = [
        bspec((B, D, DM), True),
        bspec((B, Q, DM), True),
        bspec((B, 1, D), True),
        bspec((B, 1, Q), True),
        bspec((B, D + Q, D + Q), True),
        bspec((2, DM, DM), False),
        bspec((2, DM), False),
        bspec((2, DM, DM), False),
        bspec((2, DM), False),
        bspec((2, DM, DM), False),
        bspec((2, DM), False),
        bspec((2, DM, DM), False),
        bspec((2, DM), False),
        bspec((2, DM, DM), False),
        bspec((2, DM), False),
        bspec((2, DM), False),
        bspec((8, H, DK, DK), False),
        bspec((8, H, DK, DK), False),
        bspec((2, DM), False),
        bspec((2, DM), False),
        bspec((DM, DM), False),
        bspec((1, DM), False),
    ]
    out_specs = [bspec((B, D, DM), True), bspec((B, Q, DM), True)]

    outd, outq = pl.pallas_call(
        _hgt_body,
        grid=(B,),
        in_specs=in_specs,
        out_specs=out_specs,
        out_shape=[jax.ShapeDtypeStruct((B, D, DM), f32),
                   jax.ShapeDtypeStruct((B, Q, DM), f32)],
    )(d_node, q_node, dmask3, qmask3, graph,
      adapt_W, adapt_b, Wk, bk, Wq, bq, Wv, bv, Wa_eff, ba_eff, hmix,
      A_eff, rel_msg, ln_g, ln_b, out_W, ob2)
    return outd, outq


# speedup vs baseline: 3.1797x; 1.0561x over previous
import functools

import jax
import jax.numpy as jnp
import numpy as np
from jax.experimental import pallas as pl
from jax.experimental.pallas import tpu as pltpu

B, D, Q, DM, H = 8, 512, 128, 128, 2
DK = DM // H
SQRT_DK = float(np.sqrt(DK))
EPS = 1e-5
_RELS = [(0, 1, 0), (0, 0, 1), (1, 1, 2), (1, 0, 3),
         (0, 1, 4), (0, 0, 5), (1, 1, 6), (1, 0, 7)]


def _hgt_body(dn_ref, qn_ref, dm_ref, qm_ref, g_ref,
              aW_ref, ab_ref, Wk_ref, bk_ref, Wq_ref, bq_ref, Wv_ref, bv_ref,
              Wa_ref, ba_ref, hmix_ref, Ablk_ref, Mblk_ref,
              ln_g_ref, ln_b_ref, oW_ref, ob_ref,
              outd_ref, outq_ref):
    f32 = jnp.float32

    def mm(a, b):
        return jax.lax.dot_general(a, b, (((1,), (0,)), ((), ())),
                                   preferred_element_type=f32)

    def mm_nt(a, b):
        return jax.lax.dot_general(a, b, (((1,), (1,)), ((), ())),
                                   preferred_element_type=f32)

    def mm_tn(a, b):
        return jax.lax.dot_general(a, b, (((0,), (0,)), ((), ())),
                                   preferred_element_type=f32)

    dn = dn_ref[0]
    qn = qn_ref[0]
    dmf = dm_ref[0].astype(f32)
    qmf = qm_ref[0].astype(f32)
    g = g_ref[0] > 0

    def gelu_exact(x):
        return x * 0.5 * (1.0 + jax.lax.erf(x * np.float32(1.0 / np.sqrt(2.0))))

    h = [gelu_exact(mm(dn, aW_ref[0]) + ab_ref[0][None, :]),
         gelu_exact(mm(qn, aW_ref[1]) + ab_ref[1][None, :])]
    kb = [mm(h[t], Wk_ref[t]) + bk_ref[t][None, :] for t in (0, 1)]
    qb = [mm(h[t], Wq_ref[t]) + bq_ref[t][None, :] for t in (0, 1)]
    vb = [mm(h[t], Wv_ref[t]) + bv_ref[t][None, :] for t in (0, 1)]

    eye_d = (jax.lax.broadcasted_iota(jnp.int32, (D, D), 0)
             == jax.lax.broadcasted_iota(jnp.int32, (D, D), 1))
    eye_q = (jax.lax.broadcasted_iota(jnp.int32, (Q, Q), 0)
             == jax.lax.broadcasted_iota(jnp.int32, (Q, Q), 1))
    pair = {
        (0, 0): (mm_tn(dmf, dmf) > 0) & ~eye_d,
        (1, 1): (mm_tn(qmf, qmf) > 0) & ~eye_q,
        (0, 1): mm_tn(dmf, qmf) > 0,
        (1, 0): mm_tn(qmf, dmf) > 0,
    }
    gblk = {
        (0, 0): g[:D, :D], (0, 1): g[:D, D:],
        (1, 0): g[D:, :D], (1, 1): g[D:, D:],
    }

    ones_col = {0: jnp.full((D, 1), 1.0, f32), 1: jnp.full((Q, 1), 1.0, f32)}

    agg = [None, None]
    for (s, t, e) in _RELS:
        lg = e < 4
        gm = gblk[(s, t)]
        mask = pair[(s, t)] & (gm if lg else ~gm)
        bias = jnp.where(mask, 0.0, -1e30)
        k128 = mm(kb[s], Ablk_ref[e])
        v128 = mm(vb[s], Mblk_ref[e])
        heads = []
        for hh in range(H):
            sl = slice(hh * DK, (hh + 1) * DK)
            th = mm_nt(k128[:, sl], qb[t][:, sl]) + bias
            mx = jnp.maximum(jnp.max(th, axis=0, keepdims=True), -1e29)
            p = jnp.exp(th - mx)
            v_ext = jnp.concatenate([v128[:, sl], ones_col[s]], axis=1)
            r = mm_tn(p, v_ext)
            heads.append(r[:, :DK] / jnp.maximum(r[:, DK:DK + 1], 1e-30))
        contrib = jnp.concatenate(heads, axis=1)
        agg[t] = contrib if agg[t] is None else agg[t] + contrib

    for t, out_ref in ((0, outd_ref), (1, outq_ref)):
        tt = agg[t] * 0.25
        trans = mm(tt, Wa_ref[t]) + ba_ref[t][None, :] + h[t] * hmix_ref[t][None, :]
        mu = jnp.mean(trans, axis=-1, keepdims=True)
        cent = trans - mu
        var = jnp.mean(cent * cent, axis=-1, keepdims=True)
        nh = cent * jax.lax.rsqrt(var + EPS) * ln_g_ref[t][None, :] + ln_b_ref[t][None, :]
        out_ref[0] = mm(nh, oW_ref[...]) + ob_ref[0][None, :]


@jax.jit
def kernel(d_node, q_node, d_node_mask, q_node_mask, graph,
           adapt_W, adapt_b, Wk, bk, Wq, bq, Wv, bv, Wa, ba,
           rel_pri, rel_att, rel_msg, skip, ln_g, ln_b, out_W, out_b):
    f32 = jnp.float32
    A_eff = rel_att * (rel_pri / SQRT_DK)[:, :, None, None]
    zero = jnp.zeros((8, DK, DK), f32)
    Ablk = jnp.concatenate([
        jnp.concatenate([A_eff[:, 0], zero], axis=2),
        jnp.concatenate([zero, A_eff[:, 1]], axis=2)], axis=1)
    Mblk = jnp.concatenate([
        jnp.concatenate([rel_msg[:, 0], zero], axis=2),
        jnp.concatenate([zero, rel_msg[:, 1]], axis=2)], axis=1)
    alpha = jax.nn.sigmoid(skip)
    Wa_eff = Wa * alpha[:, None, None]
    ba_eff = ba * alpha[:, None]
    hmix = jnp.broadcast_to((1.0 - alpha)[:, None], (2, DM)).astype(f32)
    dmask3 = d_node_mask.reshape(B, 1, D)
    qmask3 = q_node_mask.reshape(B, 1, Q)
    ob2 = out_b.reshape(1, DM)

    def bspec(shape, batched):
        if batched:
            return pl.BlockSpec((1,) + shape[1:],
                                lambda b: (b,) + (0,) * (len(shape) - 1))
        return pl.BlockSpec(shape, lambda b: (0,) * len(shape))

    in_specs = [
        bspec((B, D, DM), True),
        bspec((B, Q, DM), True),
        bspec((B, 1, D), True),
        bspec((B, 1, Q), True),
        bspec((B, D + Q, D + Q), True),
        bspec((2, DM, DM), False),
        bspec((2, DM), False),
        bspec((2, DM, DM), False),
        bspec((2, DM), False),
        bspec((2, DM, DM), False),
        bspec((2, DM), False),
        bspec((2, DM, DM), False),
        bspec((2, DM), False),
        bspec((2, DM, DM), False),
        bspec((2, DM), False),
        bspec((2, DM), False),
        bspec((8, DM, DM), False),
        bspec((8, DM, DM), False),
        bspec((2, DM), False),
        bspec((2, DM), False),
        bspec((DM, DM), False),
        bspec((1, DM), False),
    ]
    out_specs = [bspec((B, D, DM), True), bspec((B, Q, DM), True)]

    outd, outq = pl.pallas_call(
        _hgt_body,
        grid=(B,),
        in_specs=in_specs,
        out_specs=out_specs,
        out_shape=[jax.ShapeDtypeStruct((B, D, DM), f32),
                   jax.ShapeDtypeStruct((B, Q, DM), f32)],
        compiler_params=pltpu.CompilerParams(
            dimension_semantics=("parallel",)),
    )(d_node, q_node, dmask3, qmask3, graph,
      adapt_W, adapt_b, Wk, bk, Wq, bq, Wv, bv, Wa_eff, ba_eff, hmix,
      Ablk, Mblk, ln_g, ln_b, out_W, ob2)
    return outd, outq


# speedup vs baseline: 3.2575x; 1.0245x over previous
import functools

import jax
import jax.numpy as jnp
import numpy as np
from jax.experimental import pallas as pl
from jax.experimental.pallas import tpu as pltpu

B, D, Q, DM, H = 8, 512, 128, 128, 2
DK = DM // H
SQRT_DK = float(np.sqrt(DK))
EPS = 1e-5
_RELS = [(0, 1, 0), (0, 0, 1), (1, 1, 2), (1, 0, 3),
         (0, 1, 4), (0, 0, 5), (1, 1, 6), (1, 0, 7)]


def _hgt_body(dn_ref, qn_ref, dm_ref, qm_ref, g_ref,
              aW_ref, ab_ref, Wk_ref, bk_ref, Wq_ref, bq_ref, Wv_ref, bv_ref,
              Wa_ref, ba_ref, hmix_ref, Ablk_ref, Mblk_ref,
              ln_g_ref, ln_b_ref, oW_ref, ob_ref,
              outd_ref, outq_ref):
    f32 = jnp.float32

    def mm(a, b):
        return jax.lax.dot_general(a, b, (((1,), (0,)), ((), ())),
                                   preferred_element_type=f32)

    def mm_nt(a, b):
        return jax.lax.dot_general(a, b, (((1,), (1,)), ((), ())),
                                   preferred_element_type=f32)

    def mm_tn(a, b):
        return jax.lax.dot_general(a, b, (((0,), (0,)), ((), ())),
                                   preferred_element_type=f32)

    dn = dn_ref[0]
    qn = qn_ref[0]
    dmf = dm_ref[0].astype(f32)
    qmf = qm_ref[0].astype(f32)
    g = g_ref[0]

    def gelu_exact(x):
        return x * 0.5 * (1.0 + jax.lax.erf(x * np.float32(1.0 / np.sqrt(2.0))))

    h = [gelu_exact(mm(dn, aW_ref[0]) + ab_ref[0][None, :]),
         gelu_exact(mm(qn, aW_ref[1]) + ab_ref[1][None, :])]
    kb = [mm(h[t], Wk_ref[t]) + bk_ref[t][None, :] for t in (0, 1)]
    qb = [mm(h[t], Wq_ref[t]) + bq_ref[t][None, :] for t in (0, 1)]
    vb = [mm(h[t], Wv_ref[t]) + bv_ref[t][None, :] for t in (0, 1)]

    eye_d = jnp.where(jax.lax.broadcasted_iota(jnp.int32, (D, D), 0)
                      == jax.lax.broadcasted_iota(jnp.int32, (D, D), 1),
                      0.0, 1.0).astype(f32)
    eye_q = jnp.where(jax.lax.broadcasted_iota(jnp.int32, (Q, Q), 0)
                      == jax.lax.broadcasted_iota(jnp.int32, (Q, Q), 1),
                      0.0, 1.0).astype(f32)
    pair = {
        (0, 0): mm_tn(dmf, dmf) * eye_d,
        (1, 1): mm_tn(qmf, qmf) * eye_q,
        (0, 1): mm_tn(dmf, qmf),
        (1, 0): mm_tn(qmf, dmf),
    }
    gf = (g > 0).astype(f32)
    gcf = 1.0 - gf
    gblk = {
        (0, 0): (gf[:D, :D], gcf[:D, :D]), (0, 1): (gf[:D, D:], gcf[:D, D:]),
        (1, 0): (gf[D:, :D], gcf[D:, :D]), (1, 1): (gf[D:, D:], gcf[D:, D:]),
    }

    ones_col = {0: jnp.full((D, 1), 1.0, f32), 1: jnp.full((Q, 1), 1.0, f32)}

    aggT = [None, None]
    for (s, t, e) in _RELS:
        maskf = pair[(s, t)] * gblk[(s, t)][0 if e < 4 else 1]
        k128 = mm(kb[s], Ablk_ref[e])
        v128 = mm(vb[s], Mblk_ref[e])
        heads = []
        for hh in range(H):
            sl = slice(hh * DK, (hh + 1) * DK)
            th = mm_nt(k128[:, sl], qb[t][:, sl])
            mx = jnp.max(th, axis=0, keepdims=True)
            p = jnp.exp(th - mx) * maskf
            v_ext = jnp.concatenate([v128[:, sl], ones_col[s]], axis=1)
            r = mm_tn(v_ext, p)
            heads.append(r[:DK, :] / jnp.maximum(r[DK:DK + 1, :], 1e-30))
        contrib = jnp.concatenate(heads, axis=0)
        aggT[t] = contrib if aggT[t] is None else aggT[t] + contrib

    for t, out_ref in ((0, outd_ref), (1, outq_ref)):
        trans = (mm_tn(aggT[t], Wa_ref[t])
                 + ba_ref[t][None, :] + h[t] * hmix_ref[t][None, :])
        mu = jnp.mean(trans, axis=-1, keepdims=True)
        cent = trans - mu
        var = jnp.mean(cent * cent, axis=-1, keepdims=True)
        nh = cent * jax.lax.rsqrt(var + EPS) * ln_g_ref[t][None, :] + ln_b_ref[t][None, :]
        out_ref[0] = mm(nh, oW_ref[...]) + ob_ref[0][None, :]


@jax.jit
def kernel(d_node, q_node, d_node_mask, q_node_mask, graph,
           adapt_W, adapt_b, Wk, bk, Wq, bq, Wv, bv, Wa, ba,
           rel_pri, rel_att, rel_msg, skip, ln_g, ln_b, out_W, out_b):
    f32 = jnp.float32
    A_eff = rel_att * (rel_pri / SQRT_DK)[:, :, None, None]
    zero = jnp.zeros((8, DK, DK), f32)
    Ablk = jnp.concatenate([
        jnp.concatenate([A_eff[:, 0], zero], axis=2),
        jnp.concatenate([zero, A_eff[:, 1]], axis=2)], axis=1)
    Mblk = jnp.concatenate([
        jnp.concatenate([rel_msg[:, 0], zero], axis=2),
        jnp.concatenate([zero, rel_msg[:, 1]], axis=2)], axis=1)
    alpha = jax.nn.sigmoid(skip)
    Wa_eff = Wa * (0.25 * alpha)[:, None, None]
    ba_eff = ba * alpha[:, None]
    hmix = jnp.broadcast_to((1.0 - alpha)[:, None], (2, DM)).astype(f32)
    dmask3 = d_node_mask.reshape(B, 1, D)
    qmask3 = q_node_mask.reshape(B, 1, Q)
    ob2 = out_b.reshape(1, DM)

    def bspec(shape, batched):
        if batched:
            return pl.BlockSpec((1,) + shape[1:],
                                lambda b: (b,) + (0,) * (len(shape) - 1))
        return pl.BlockSpec(shape, lambda b: (0,) * len(shape))

    in_specs = [
        bspec((B, D, DM), True),
        bspec((B, Q, DM), True),
        bspec((B, 1, D), True),
        bspec((B, 1, Q), True),
        bspec((B, D + Q, D + Q), True),
        bspec((2, DM, DM), False),
        bspec((2, DM), False),
        bspec((2, DM, DM), False),
        bspec((2, DM), False),
        bspec((2, DM, DM), False),
        bspec((2, DM), False),
        bspec((2, DM, DM), False),
        bspec((2, DM), False),
        bspec((2, DM, DM), False),
        bspec((2, DM), False),
        bspec((2, DM), False),
        bspec((8, DM, DM), False),
        bspec((8, DM, DM), False),
        bspec((2, DM), False),
        bspec((2, DM), False),
        bspec((DM, DM), False),
        bspec((1, DM), False),
    ]
    out_specs = [bspec((B, D, DM), True), bspec((B, Q, DM), True)]

    outd, outq = pl.pallas_call(
        _hgt_body,
        grid=(B,),
        in_specs=in_specs,
        out_specs=out_specs,
        out_shape=[jax.ShapeDtypeStruct((B, D, DM), f32),
                   jax.ShapeDtypeStruct((B, Q, DM), f32)],
        compiler_params=pltpu.CompilerParams(
            dimension_semantics=("parallel",)),
    )(d_node, q_node, dmask3, qmask3, graph,
      adapt_W, adapt_b, Wk, bk, Wq, bq, Wv, bv, Wa_eff, ba_eff, hmix,
      Ablk, Mblk, ln_g, ln_b, out_W, ob2)
    return outd, outq


# speedup vs baseline: 3.4985x; 1.0740x over previous
import jax
import jax.numpy as jnp
import numpy as np
from jax.experimental import pallas as pl
from jax.experimental.pallas import tpu as pltpu

B, D, Q, DM, H = 8, 512, 128, 128, 2
DK = DM // H
SQRT_DK = float(np.sqrt(DK))
EPS = 1e-5
_RELS = [(0, 1, 0), (0, 0, 1), (1, 1, 2), (1, 0, 3),
         (0, 1, 4), (0, 0, 5), (1, 1, 6), (1, 0, 7)]


def _hgt_body(dn_ref, qn_ref, dm_ref, qm_ref, g_ref,
              aW_ref, ab_ref, Wk_ref, bk_ref, Wq_ref, bq_ref, Wv_ref, bv_ref,
              Wa_ref, ba_ref, pri_ref, A_ref, M_ref, skip_ref,
              ln_g_ref, ln_b_ref, oW_ref, ob_ref,
              outd_ref, outq_ref):
    f32 = jnp.float32

    def mm(a, b):
        return jax.lax.dot_general(a, b, (((1,), (0,)), ((), ())),
                                   preferred_element_type=f32)

    def mm_nt(a, b):
        return jax.lax.dot_general(a, b, (((1,), (1,)), ((), ())),
                                   preferred_element_type=f32)

    def mm_tn(a, b):
        return jax.lax.dot_general(a, b, (((0,), (0,)), ((), ())),
                                   preferred_element_type=f32)

    dn = dn_ref[0]
    qn = qn_ref[0]
    dmf = dm_ref[0].astype(f32)
    qmf = qm_ref[0].astype(f32)
    g = g_ref[0]

    def gelu_exact(x):
        return x * 0.5 * (1.0 + jax.lax.erf(x * np.float32(1.0 / np.sqrt(2.0))))

    h = [gelu_exact(mm(dn, aW_ref[0]) + ab_ref[0][None, :]),
         gelu_exact(mm(qn, aW_ref[1]) + ab_ref[1][None, :])]
    kb = [mm(h[t], Wk_ref[t]) + bk_ref[t][None, :] for t in (0, 1)]
    qb = [mm(h[t], Wq_ref[t]) + bq_ref[t][None, :] for t in (0, 1)]
    vb = [mm(h[t], Wv_ref[t]) + bv_ref[t][None, :] for t in (0, 1)]

    eye_d = jnp.where(jax.lax.broadcasted_iota(jnp.int32, (D, D), 0)
                      == jax.lax.broadcasted_iota(jnp.int32, (D, D), 1),
                      0.0, 1.0).astype(f32)
    eye_q = jnp.where(jax.lax.broadcasted_iota(jnp.int32, (Q, Q), 0)
                      == jax.lax.broadcasted_iota(jnp.int32, (Q, Q), 1),
                      0.0, 1.0).astype(f32)
    pair = {
        (0, 0): mm_tn(dmf, dmf) * eye_d,
        (1, 1): mm_tn(qmf, qmf) * eye_q,
        (0, 1): mm_tn(dmf, qmf),
        (1, 0): mm_tn(qmf, dmf),
    }
    gf = (g > 0).astype(f32)
    gcf = 1.0 - gf
    gblk = {
        (0, 0): (gf[:D, :D], gcf[:D, :D]), (0, 1): (gf[:D, D:], gcf[:D, D:]),
        (1, 0): (gf[D:, :D], gcf[D:, :D]), (1, 1): (gf[D:, D:], gcf[D:, D:]),
    }

    z64 = jnp.zeros((DK, DK), f32)

    def blockdiag(m0, m1):
        return jnp.concatenate(
            [jnp.concatenate([m0, z64], axis=1),
             jnp.concatenate([z64, m1], axis=1)], axis=0)

    ones_col = {0: jnp.full((D, 1), 1.0, f32), 1: jnp.full((Q, 1), 1.0, f32)}

    aggT = [None, None]
    for (s, t, e) in _RELS:
        maskf = pair[(s, t)] * gblk[(s, t)][0 if e < 4 else 1]
        scale = np.float32(1.0 / SQRT_DK)
        ablk = blockdiag(A_ref[e, 0] * (pri_ref[e:e + 1, 0:1] * scale),
                         A_ref[e, 1] * (pri_ref[e:e + 1, 1:2] * scale))
        mblk = blockdiag(M_ref[e, 0], M_ref[e, 1])
        k128 = mm(kb[s], ablk)
        v128 = mm(vb[s], mblk)
        heads = []
        for hh in range(H):
            sl = slice(hh * DK, (hh + 1) * DK)
            th = mm_nt(k128[:, sl], qb[t][:, sl])
            mx = jnp.max(th, axis=0, keepdims=True)
            p = jnp.exp(th - mx) * maskf
            v_ext = jnp.concatenate([v128[:, sl], ones_col[s]], axis=1)
            r = mm_tn(v_ext, p)
            heads.append(r[:DK, :] / jnp.maximum(r[DK:DK + 1, :], 1e-30))
        contrib = jnp.concatenate(heads, axis=0)
        aggT[t] = contrib if aggT[t] is None else aggT[t] + contrib

    alpha = jax.nn.sigmoid(skip_ref[...])
    for t, out_ref in ((0, outd_ref), (1, outq_ref)):
        a_t = alpha[0:1, t:t + 1]
        trans = (mm_tn(aggT[t], Wa_ref[t]) * (0.25 * a_t)
                 + ba_ref[t][None, :] * a_t + h[t] * (1.0 - a_t))
        mu = jnp.mean(trans, axis=-1, keepdims=True)
        cent = trans - mu
        var = jnp.mean(cent * cent, axis=-1, keepdims=True)
        nh = cent * jax.lax.rsqrt(var + EPS) * ln_g_ref[t][None, :] + ln_b_ref[t][None, :]
        out_ref[0] = mm(nh, oW_ref[...]) + ob_ref[0][None, :]


@jax.jit
def kernel(d_node, q_node, d_node_mask, q_node_mask, graph,
           adapt_W, adapt_b, Wk, bk, Wq, bq, Wv, bv, Wa, ba,
           rel_pri, rel_att, rel_msg, skip, ln_g, ln_b, out_W, out_b):
    f32 = jnp.float32
    dmask3 = d_node_mask.reshape(B, 1, D)
    qmask3 = q_node_mask.reshape(B, 1, Q)
    skip2 = skip.reshape(1, 2)
    ob2 = out_b.reshape(1, DM)

    def bspec(shape, batched):
        if batched:
            return pl.BlockSpec((1,) + shape[1:],
                                lambda b: (b,) + (0,) * (len(shape) - 1))
        return pl.BlockSpec(shape, lambda b: (0,) * len(shape))

    in_specs = [
        bspec((B, D, DM), True),
        bspec((B, Q, DM), True),
        bspec((B, 1, D), True),
        bspec((B, 1, Q), True),
        bspec((B, D + Q, D + Q), True),
        bspec((2, DM, DM), False),
        bspec((2, DM), False),
        bspec((2, DM, DM), False),
        bspec((2, DM), False),
        bspec((2, DM, DM), False),
        bspec((2, DM), False),
        bspec((2, DM, DM), False),
        bspec((2, DM), False),
        bspec((2, DM, DM), False),
        bspec((2, DM), False),
        bspec((8, H), False),
        bspec((8, H, DK, DK), False),
        bspec((8, H, DK, DK), False),
        bspec((1, 2), False),
        bspec((2, DM), False),
        bspec((2, DM), False),
        bspec((DM, DM), False),
        bspec((1, DM), False),
    ]
    out_specs = [bspec((B, D, DM), True), bspec((B, Q, DM), True)]

    outd, outq = pl.pallas_call(
        _hgt_body,
        grid=(B,),
        in_specs=in_specs,
        out_specs=out_specs,
        out_shape=[jax.ShapeDtypeStruct((B, D, DM), f32),
                   jax.ShapeDtypeStruct((B, Q, DM), f32)],
        compiler_params=pltpu.CompilerParams(
            dimension_semantics=("parallel",)),
    )(d_node, q_node, dmask3, qmask3, graph,
      adapt_W, adapt_b, Wk, bk, Wq, bq, Wv, bv, Wa, ba,
      rel_pri, rel_att, rel_msg, skip2, ln_g, ln_b, out_W, ob2)
    return outd, outq


# speedup vs baseline: 3.5696x; 1.0203x over previous
import jax
import jax.numpy as jnp
import numpy as np
from jax.experimental import pallas as pl
from jax.experimental.pallas import tpu as pltpu

B, D, Q, DM, H = 8, 512, 128, 128, 2
DK = DM // H
SQRT_DK = float(np.sqrt(DK))
EPS = 1e-5
NB = 2
_RELS = [(0, 1, 0), (0, 0, 1), (1, 1, 2), (1, 0, 3),
         (0, 1, 4), (0, 0, 5), (1, 1, 6), (1, 0, 7)]


def _hgt_body(dn_ref, qn_ref, dm_ref, qm_ref, g_ref,
              aW_ref, ab_ref, Wk_ref, bk_ref, Wq_ref, bq_ref, Wv_ref, bv_ref,
              Wa_ref, ba_ref, pri_ref, A_ref, M_ref, skip_ref,
              ln_g_ref, ln_b_ref, oW_ref, ob_ref,
              outd_ref, outq_ref):
    f32 = jnp.float32
    bf16 = jnp.bfloat16

    def mm(a, b):
        return jax.lax.dot_general(a, b, (((1,), (0,)), ((), ())),
                                   preferred_element_type=f32)

    def mm_nt(a, b):
        return jax.lax.dot_general(a, b, (((1,), (1,)), ((), ())),
                                   preferred_element_type=f32)

    def mm_tn(a, b):
        return jax.lax.dot_general(a, b, (((0,), (0,)), ((), ())),
                                   preferred_element_type=f32)

    def gelu_exact(x):
        return x * 0.5 * (1.0 + jax.lax.erf(x * np.float32(1.0 / np.sqrt(2.0))))

    eye_d = jnp.where(jax.lax.broadcasted_iota(jnp.int32, (D, D), 0)
                      == jax.lax.broadcasted_iota(jnp.int32, (D, D), 1),
                      0.0, 1.0).astype(f32)
    eye_q = jnp.where(jax.lax.broadcasted_iota(jnp.int32, (Q, Q), 0)
                      == jax.lax.broadcasted_iota(jnp.int32, (Q, Q), 1),
                      0.0, 1.0).astype(f32)
    z64 = jnp.zeros((DK, DK), f32)

    def blockdiag(m0, m1):
        return jnp.concatenate(
            [jnp.concatenate([m0, z64], axis=1),
             jnp.concatenate([z64, m1], axis=1)], axis=0)

    scale = np.float32(1.0 / SQRT_DK)
    ablk = [blockdiag(A_ref[e, 0] * (pri_ref[e:e + 1, 0:1] * scale),
                      A_ref[e, 1] * (pri_ref[e:e + 1, 1:2] * scale))
            for e in range(8)]
    mblk = [blockdiag(M_ref[e, 0], M_ref[e, 1]) for e in range(8)]
    ones_col = {0: jnp.full((D, 1), 1.0, f32), 1: jnp.full((Q, 1), 1.0, f32)}
    alpha = jax.nn.sigmoid(skip_ref[...])

    for bi in range(NB):
        dmf = dm_ref[bi].astype(f32)
        qmf = qm_ref[bi].astype(f32)
        g = g_ref[bi]

        h = [gelu_exact(mm(dn_ref[bi], aW_ref[0]) + ab_ref[0][None, :]),
             gelu_exact(mm(qn_ref[bi], aW_ref[1]) + ab_ref[1][None, :])]
        kb = [mm(h[t], Wk_ref[t]) + bk_ref[t][None, :] for t in (0, 1)]
        qb = [mm(h[t], Wq_ref[t]) + bq_ref[t][None, :] for t in (0, 1)]
        vb = [mm(h[t], Wv_ref[t]) + bv_ref[t][None, :] for t in (0, 1)]

        pair = {
            (0, 0): mm_tn(dmf, dmf) * eye_d,
            (1, 1): mm_tn(qmf, qmf) * eye_q,
            (0, 1): mm_tn(dmf, qmf),
            (1, 0): mm_tn(qmf, dmf),
        }
        gf = (g > 0).astype(f32)
        gcf = 1.0 - gf
        gblk = {
            (0, 0): (gf[:D, :D], gcf[:D, :D]), (0, 1): (gf[:D, D:], gcf[:D, D:]),
            (1, 0): (gf[D:, :D], gcf[D:, :D]), (1, 1): (gf[D:, D:], gcf[D:, D:]),
        }

        aggT = [None, None]
        for (s, t, e) in _RELS:
            maskf = pair[(s, t)] * gblk[(s, t)][0 if e < 4 else 1]
            k128 = mm(kb[s], ablk[e])
            v128 = mm(vb[s], mblk[e])
            heads = []
            for hh in range(H):
                sl = slice(hh * DK, (hh + 1) * DK)
                th = mm_nt(k128[:, sl], qb[t][:, sl])
                mx = jnp.max(th, axis=0, keepdims=True)
                p = jnp.exp(th - mx) * maskf
                v_ext = jnp.concatenate([v128[:, sl], ones_col[s]], axis=1)
                r = mm_tn(v_ext.astype(bf16), p.astype(bf16))
                heads.append(r[:DK, :] / jnp.maximum(r[DK:DK + 1, :], 1e-30))
            contrib = jnp.concatenate(heads, axis=0)
            aggT[t] = contrib if aggT[t] is None else aggT[t] + contrib

        for t, out_ref in ((0, outd_ref), (1, outq_ref)):
            a_t = alpha[0:1, t:t + 1]
            trans = (mm_tn(aggT[t], Wa_ref[t]) * (0.25 * a_t)
                     + ba_ref[t][None, :] * a_t + h[t] * (1.0 - a_t))
            mu = jnp.mean(trans, axis=-1, keepdims=True)
            cent = trans - mu
            var = jnp.mean(cent * cent, axis=-1, keepdims=True)
            nh = (cent * jax.lax.rsqrt(var + EPS) * ln_g_ref[t][None, :]
                  + ln_b_ref[t][None, :])
            out_ref[bi] = mm(nh, oW_ref[...]) + ob_ref[0][None, :]


@jax.jit
def kernel(d_node, q_node, d_node_mask, q_node_mask, graph,
           adapt_W, adapt_b, Wk, bk, Wq, bq, Wv, bv, Wa, ba,
           rel_pri, rel_att, rel_msg, skip, ln_g, ln_b, out_W, out_b):
    f32 = jnp.float32
    dmask3 = d_node_mask.reshape(B, 1, D)
    qmask3 = q_node_mask.reshape(B, 1, Q)
    skip2 = skip.reshape(1, 2)
    ob2 = out_b.reshape(1, DM)

    def bspec(shape, batched):
        if batched:
            return pl.BlockSpec((NB,) + shape[1:],
                                lambda b: (b,) + (0,) * (len(shape) - 1))
        return pl.BlockSpec(shape, lambda b: (0,) * len(shape))

    in_specs = [
        bspec((B, D, DM), True),
        bspec((B, Q, DM), True),
        bspec((B, 1, D), True),
        bspec((B, 1, Q), True),
        bspec((B, D + Q, D + Q), True),
        bspec((2, DM, DM), False),
        bspec((2, DM), False),
        bspec((2, DM, DM), False),
        bspec((2, DM), False),
        bspec((2, DM, DM), False),
        bspec((2, DM), False),
        bspec((2, DM, DM), False),
        bspec((2, DM), False),
        bspec((2, DM, DM), False),
        bspec((2, DM), False),
        bspec((8, H), False),
        bspec((8, H, DK, DK), False),
        bspec((8, H, DK, DK), False),
        bspec((1, 2), False),
        bspec((2, DM), False),
        bspec((2, DM), False),
        bspec((DM, DM), False),
        bspec((1, DM), False),
    ]
    out_specs = [bspec((B, D, DM), True), bspec((B, Q, DM), True)]

    outd, outq = pl.pallas_call(
        _hgt_body,
        grid=(B // NB,),
        in_specs=in_specs,
        out_specs=out_specs,
        out_shape=[jax.ShapeDtypeStruct((B, D, DM), f32),
                   jax.ShapeDtypeStruct((B, Q, DM), f32)],
        compiler_params=pltpu.CompilerParams(
            dimension_semantics=("parallel",)),
    )(d_node, q_node, dmask3, qmask3, graph,
      adapt_W, adapt_b, Wk, bk, Wq, bq, Wv, bv, Wa, ba,
      rel_pri, rel_att, rel_msg, skip2, ln_g, ln_b, out_W, ob2)
    return outd, outq


# speedup vs baseline: 5.7888x; 1.6217x over previous
import jax
import jax.numpy as jnp
import numpy as np
from jax.experimental import pallas as pl
from jax.experimental.pallas import tpu as pltpu

B, D, Q, DM, H = 8, 512, 128, 128, 2
DK = DM // H
SQRT_DK = float(np.sqrt(DK))
EPS = 1e-5
NB = 2
_RELS = [(0, 1, 0), (0, 0, 1), (1, 1, 2), (1, 0, 3),
         (0, 1, 4), (0, 0, 5), (1, 1, 6), (1, 0, 7)]


def _hgt_body(dn_ref, qn_ref, dm_ref, qm_ref, g_ref,
              aW_ref, ab_ref, Wk_ref, bk_ref, Wq_ref, bq_ref, Wv_ref, bv_ref,
              Wa_ref, ba_ref, pri_ref, A_ref, M_ref, skip_ref,
              ln_g_ref, ln_b_ref, oW_ref, ob_ref,
              outd_ref, outq_ref):
    f32 = jnp.float32
    bf16 = jnp.bfloat16

    def mm(a, b):
        return jax.lax.dot_general(a, b, (((1,), (0,)), ((), ())),
                                   preferred_element_type=f32)

    def mm_nt(a, b):
        return jax.lax.dot_general(a, b, (((1,), (1,)), ((), ())),
                                   preferred_element_type=f32)

    def mm_tn(a, b):
        return jax.lax.dot_general(a, b, (((0,), (0,)), ((), ())),
                                   preferred_element_type=f32)

    def gelu_exact(x):
        return x * 0.5 * (1.0 + jax.lax.erf(x * np.float32(1.0 / np.sqrt(2.0))))

    eye_d = jnp.where(jax.lax.broadcasted_iota(jnp.int32, (D, D), 0)
                      == jax.lax.broadcasted_iota(jnp.int32, (D, D), 1),
                      0.0, 1.0).astype(f32)
    eye_q = jnp.where(jax.lax.broadcasted_iota(jnp.int32, (Q, Q), 0)
                      == jax.lax.broadcasted_iota(jnp.int32, (Q, Q), 1),
                      0.0, 1.0).astype(f32)
    z64 = jnp.zeros((DK, DK), f32)

    def blockdiag(m0, m1):
        return jnp.concatenate(
            [jnp.concatenate([m0, z64], axis=1),
             jnp.concatenate([z64, m1], axis=1)], axis=0)

    scale = np.float32(1.0 / SQRT_DK)
    ablk = [blockdiag(A_ref[e, 0] * (pri_ref[e:e + 1, 0:1] * scale),
                      A_ref[e, 1] * (pri_ref[e:e + 1, 1:2] * scale))
            for e in range(8)]
    mblk = [blockdiag(M_ref[e, 0], M_ref[e, 1]) for e in range(8)]
    ones_col = {0: jnp.full((D, 1), 1.0, f32), 1: jnp.full((Q, 1), 1.0, f32)}
    alpha = jax.nn.sigmoid(skip_ref[...])

    for bi in range(NB):
        dmf = dm_ref[bi].astype(f32)
        qmf = qm_ref[bi].astype(f32)
        g = g_ref[bi]

        h = [gelu_exact(mm(dn_ref[bi], aW_ref[0]) + ab_ref[0][None, :]),
             gelu_exact(mm(qn_ref[bi], aW_ref[1]) + ab_ref[1][None, :])]
        kb = [mm(h[t], Wk_ref[t]) + bk_ref[t][None, :] for t in (0, 1)]
        qb = [mm(h[t], Wq_ref[t]) + bq_ref[t][None, :] for t in (0, 1)]
        vb = [mm(h[t], Wv_ref[t]) + bv_ref[t][None, :] for t in (0, 1)]

        pair = {
            (0, 0): mm_tn(dmf, dmf) * eye_d,
            (1, 1): mm_tn(qmf, qmf) * eye_q,
            (0, 1): mm_tn(dmf, qmf),
            (1, 0): mm_tn(qmf, dmf),
        }
        gf = (g > 0).astype(f32)
        gcf = 1.0 - gf
        gblk = {
            (0, 0): (gf[:D, :D], gcf[:D, :D]), (0, 1): (gf[:D, D:], gcf[:D, D:]),
            (1, 0): (gf[D:, :D], gcf[D:, :D]), (1, 1): (gf[D:, D:], gcf[D:, D:]),
        }

        sls = [slice(hh * DK, (hh + 1) * DK) for hh in range(H)]
        maskfs = {e: pair[(s, t)] * gblk[(s, t)][0 if e < 4 else 1]
                  for (s, t, e) in _RELS}
        k128s = {e: mm(kb[s], ablk[e]) for (s, t, e) in _RELS}
        v128s = {e: mm(vb[s], mblk[e]) for (s, t, e) in _RELS}
        units = [(s, t, e, sl) for (s, t, e) in _RELS for sl in sls]
        ths = [mm_nt(k128s[e][:, sl], qb[t][:, sl]) for (s, t, e, sl) in units]
        mxs = [jnp.max(th, axis=0, keepdims=True) for th in ths]
        ps = [jnp.exp(th - mx) * maskfs[e]
              for th, mx, (s, t, e, sl) in zip(ths, mxs, units)]
        vxs = [jnp.concatenate([v128s[e][:, sl], ones_col[s]], axis=1)
               for (s, t, e, sl) in units]
        rs = [mm_tn(vx.astype(bf16), p.astype(bf16))
              for vx, p in zip(vxs, ps)]
        heads = [r[:DK, :] / jnp.maximum(r[DK:DK + 1, :], 1e-30) for r in rs]
        aggT = [None, None]
        for i, (s, t, e, sl) in enumerate(units):
            if i % H == 0:
                contrib = jnp.concatenate(heads[i:i + H], axis=0)
                aggT[t] = contrib if aggT[t] is None else aggT[t] + contrib

        for t, out_ref in ((0, outd_ref), (1, outq_ref)):
            a_t = alpha[0:1, t:t + 1]
            trans = (mm_tn(aggT[t], Wa_ref[t]) * (0.25 * a_t)
                     + ba_ref[t][None, :] * a_t + h[t] * (1.0 - a_t))
            mu = jnp.mean(trans, axis=-1, keepdims=True)
            cent = trans - mu
            var = jnp.mean(cent * cent, axis=-1, keepdims=True)
            nh = (cent * jax.lax.rsqrt(var + EPS) * ln_g_ref[t][None, :]
                  + ln_b_ref[t][None, :])
            out_ref[bi] = mm(nh, oW_ref[...]) + ob_ref[0][None, :]


@jax.jit
def kernel(d_node, q_node, d_node_mask, q_node_mask, graph,
           adapt_W, adapt_b, Wk, bk, Wq, bq, Wv, bv, Wa, ba,
           rel_pri, rel_att, rel_msg, skip, ln_g, ln_b, out_W, out_b):
    f32 = jnp.float32
    dmask3 = d_node_mask.reshape(B, 1, D)
    qmask3 = q_node_mask.reshape(B, 1, Q)
    skip2 = skip.reshape(1, 2)
    ob2 = out_b.reshape(1, DM)

    def bspec(shape, batched):
        if batched:
            return pl.BlockSpec((NB,) + shape[1:],
                                lambda b: (b,) + (0,) * (len(shape) - 1))
        return pl.BlockSpec(shape, lambda b: (0,) * len(shape))

    in_specs = [
        bspec((B, D, DM), True),
        bspec((B, Q, DM), True),
        bspec((B, 1, D), True),
        bspec((B, 1, Q), True),
        bspec((B, D + Q, D + Q), True),
        bspec((2, DM, DM), False),
        bspec((2, DM), False),
        bspec((2, DM, DM), False),
        bspec((2, DM), False),
        bspec((2, DM, DM), False),
        bspec((2, DM), False),
        bspec((2, DM, DM), False),
        bspec((2, DM), False),
        bspec((2, DM, DM), False),
        bspec((2, DM), False),
        bspec((8, H), False),
        bspec((8, H, DK, DK), False),
        bspec((8, H, DK, DK), False),
        bspec((1, 2), False),
        bspec((2, DM), False),
        bspec((2, DM), False),
        bspec((DM, DM), False),
        bspec((1, DM), False),
    ]
    out_specs = [bspec((B, D, DM), True), bspec((B, Q, DM), True)]

    outd, outq = pl.pallas_call(
        _hgt_body,
        grid=(B // NB,),
        in_specs=in_specs,
        out_specs=out_specs,
        out_shape=[jax.ShapeDtypeStruct((B, D, DM), f32),
                   jax.ShapeDtypeStruct((B, Q, DM), f32)],
        compiler_params=pltpu.CompilerParams(
            dimension_semantics=("parallel",)),
    )(d_node, q_node, dmask3, qmask3, graph,
      adapt_W, adapt_b, Wk, bk, Wq, bq, Wv, bv, Wa, ba,
      rel_pri, rel_att, rel_msg, skip2, ln_g, ln_b, out_W, ob2)
    return outd, outq


# speedup vs baseline: 6.4642x; 1.1167x over previous
import jax
import jax.numpy as jnp
import numpy as np
from jax.experimental import pallas as pl
from jax.experimental.pallas import tpu as pltpu

B, D, Q, DM, H = 8, 512, 128, 128, 2
DK = DM // H
SQRT_DK = float(np.sqrt(DK))
EPS = 1e-5
NB = 2
_RELS = [(0, 1, 0), (0, 0, 1), (1, 1, 2), (1, 0, 3),
         (0, 1, 4), (0, 0, 5), (1, 1, 6), (1, 0, 7)]


def _hgt_body(dn_ref, qn_ref, dm_ref, qm_ref, g_ref,
              aW_ref, ab_ref, Wk_ref, bk_ref, Wq_ref, bq_ref, Wv_ref, bv_ref,
              Wa_ref, ba_ref, pri_ref, A_ref, M_ref, skip_ref,
              ln_g_ref, ln_b_ref, oW_ref, ob_ref,
              outd_ref, outq_ref):
    f32 = jnp.float32
    bf16 = jnp.bfloat16

    def mm(a, b):
        return jax.lax.dot_general(a, b, (((1,), (0,)), ((), ())),
                                   preferred_element_type=f32)

    def mm_nt(a, b):
        return jax.lax.dot_general(a, b, (((1,), (1,)), ((), ())),
                                   preferred_element_type=f32)

    def mm_tn(a, b):
        return jax.lax.dot_general(a, b, (((0,), (0,)), ((), ())),
                                   preferred_element_type=f32)

    def gelu_exact(x):
        return x * 0.5 * (1.0 + jax.lax.erf(x * np.float32(1.0 / np.sqrt(2.0))))

    eye_d = jnp.where(jax.lax.broadcasted_iota(jnp.int32, (D, D), 0)
                      == jax.lax.broadcasted_iota(jnp.int32, (D, D), 1),
                      0.0, 1.0).astype(f32)
    eye_q = jnp.where(jax.lax.broadcasted_iota(jnp.int32, (Q, Q), 0)
                      == jax.lax.broadcasted_iota(jnp.int32, (Q, Q), 1),
                      0.0, 1.0).astype(f32)
    z64 = jnp.zeros((DK, DK), f32)

    def blockdiag(m0, m1):
        return jnp.concatenate(
            [jnp.concatenate([m0, z64], axis=1),
             jnp.concatenate([z64, m1], axis=1)], axis=0)

    scale = np.float32(np.log2(np.e) / SQRT_DK)
    ablk = [blockdiag(A_ref[e, 0] * (pri_ref[e:e + 1, 0:1] * scale),
                      A_ref[e, 1] * (pri_ref[e:e + 1, 1:2] * scale))
            for e in range(8)]
    mblk = [blockdiag(M_ref[e, 0], M_ref[e, 1]) for e in range(8)]
    ones_col = {0: jnp.full((D, 1), 1.0, f32), 1: jnp.full((Q, 1), 1.0, f32)}
    alpha = jax.nn.sigmoid(skip_ref[...])

    for bi in range(NB):
        dmf = dm_ref[bi].astype(f32)
        qmf = qm_ref[bi].astype(f32)
        g = g_ref[bi]

        h = [gelu_exact(mm(dn_ref[bi], aW_ref[0]) + ab_ref[0][None, :]),
             gelu_exact(mm(qn_ref[bi], aW_ref[1]) + ab_ref[1][None, :])]
        kb = [mm(h[t], Wk_ref[t]) + bk_ref[t][None, :] for t in (0, 1)]
        qb = [mm(h[t], Wq_ref[t]) + bq_ref[t][None, :] for t in (0, 1)]
        vb = [mm(h[t], Wv_ref[t]) + bv_ref[t][None, :] for t in (0, 1)]

        pair = {
            (0, 0): mm_tn(dmf, dmf) * eye_d,
            (1, 1): mm_tn(qmf, qmf) * eye_q,
            (0, 1): mm_tn(dmf, qmf),
            (1, 0): mm_tn(qmf, dmf),
        }
        gf = (g > 0).astype(f32)
        gcf = 1.0 - gf
        gblk = {
            (0, 0): (gf[:D, :D], gcf[:D, :D]), (0, 1): (gf[:D, D:], gcf[:D, D:]),
            (1, 0): (gf[D:, :D], gcf[D:, :D]), (1, 1): (gf[D:, D:], gcf[D:, D:]),
        }

        sls = [slice(hh * DK, (hh + 1) * DK) for hh in range(H)]
        maskfs = {e: (pair[(s, t)] * gblk[(s, t)][0 if e < 4 else 1]).astype(bf16)
                  for (s, t, e) in _RELS}
        k128s = {e: mm(kb[s], ablk[e]) for (s, t, e) in _RELS}
        v128s = {e: mm(vb[s], mblk[e]).astype(bf16) for (s, t, e) in _RELS}
        units = [(s, t, e, sl) for (s, t, e) in _RELS for sl in sls]
        ths = [mm_nt(k128s[e][:, sl], qb[t][:, sl]) for (s, t, e, sl) in units]
        mxs = [jnp.max(th, axis=0, keepdims=True) for th in ths]
        ps = [jnp.exp2(th - mx).astype(bf16) * maskfs[e]
              for th, mx, (s, t, e, sl) in zip(ths, mxs, units)]
        vxs = [jnp.concatenate([v128s[e][:, sl], ones_col[s].astype(bf16)],
                               axis=1)
               for (s, t, e, sl) in units]
        rs = [mm_tn(vx, p) for vx, p in zip(vxs, ps)]
        heads = [r[:DK, :] / jnp.maximum(r[DK:DK + 1, :], 1e-30) for r in rs]
        aggT = [None, None]
        for i, (s, t, e, sl) in enumerate(units):
            if i % H == 0:
                contrib = jnp.concatenate(heads[i:i + H], axis=0)
                aggT[t] = contrib if aggT[t] is None else aggT[t] + contrib

        for t, out_ref in ((0, outd_ref), (1, outq_ref)):
            a_t = alpha[0:1, t:t + 1]
            trans = (mm_tn(aggT[t], Wa_ref[t]) * (0.25 * a_t)
                     + ba_ref[t][None, :] * a_t + h[t] * (1.0 - a_t))
            mu = jnp.mean(trans, axis=-1, keepdims=True)
            cent = trans - mu
            var = jnp.mean(cent * cent, axis=-1, keepdims=True)
            nh = (cent * jax.lax.rsqrt(var + EPS) * ln_g_ref[t][None, :]
                  + ln_b_ref[t][None, :])
            out_ref[bi] = mm(nh, oW_ref[...]) + ob_ref[0][None, :]


@jax.jit
def kernel(d_node, q_node, d_node_mask, q_node_mask, graph,
           adapt_W, adapt_b, Wk, bk, Wq, bq, Wv, bv, Wa, ba,
           rel_pri, rel_att, rel_msg, skip, ln_g, ln_b, out_W, out_b):
    f32 = jnp.float32
    dmask3 = d_node_mask.reshape(B, 1, D)
    qmask3 = q_node_mask.reshape(B, 1, Q)
    skip2 = skip.reshape(1, 2)
    ob2 = out_b.reshape(1, DM)

    def bspec(shape, batched):
        if batched:
            return pl.BlockSpec((NB,) + shape[1:],
                                lambda b: (b,) + (0,) * (len(shape) - 1))
        return pl.BlockSpec(shape, lambda b: (0,) * len(shape))

    in_specs = [
        bspec((B, D, DM), True),
        bspec((B, Q, DM), True),
        bspec((B, 1, D), True),
        bspec((B, 1, Q), True),
        bspec((B, D + Q, D + Q), True),
        bspec((2, DM, DM), False),
        bspec((2, DM), False),
        bspec((2, DM, DM), False),
        bspec((2, DM), False),
        bspec((2, DM, DM), False),
        bspec((2, DM), False),
        bspec((2, DM, DM), False),
        bspec((2, DM), False),
        bspec((2, DM, DM), False),
        bspec((2, DM), False),
        bspec((8, H), False),
        bspec((8, H, DK, DK), False),
        bspec((8, H, DK, DK), False),
        bspec((1, 2), False),
        bspec((2, DM), False),
        bspec((2, DM), False),
        bspec((DM, DM), False),
        bspec((1, DM), False),
    ]
    out_specs = [bspec((B, D, DM), True), bspec((B, Q, DM), True)]

    outd, outq = pl.pallas_call(
        _hgt_body,
        grid=(B // NB,),
        in_specs=in_specs,
        out_specs=out_specs,
        out_shape=[jax.ShapeDtypeStruct((B, D, DM), f32),
                   jax.ShapeDtypeStruct((B, Q, DM), f32)],
        compiler_params=pltpu.CompilerParams(
            dimension_semantics=("parallel",)),
    )(d_node, q_node, dmask3, qmask3, graph,
      adapt_W, adapt_b, Wk, bk, Wq, bq, Wv, bv, Wa, ba,
      rel_pri, rel_att, rel_msg, skip2, ln_g, ln_b, out_W, ob2)
    return outd, outq


# speedup vs baseline: 6.5616x; 1.0151x over previous
import jax
import jax.numpy as jnp
import numpy as np
from jax.experimental import pallas as pl
from jax.experimental.pallas import tpu as pltpu

B, D, Q, DM, H = 8, 512, 128, 128, 2
DK = DM // H
SQRT_DK = float(np.sqrt(DK))
EPS = 1e-5
NB = 2
_RELS = [(0, 1, 0), (0, 0, 1), (1, 1, 2), (1, 0, 3),
         (0, 1, 4), (0, 0, 5), (1, 1, 6), (1, 0, 7)]


def _hgt_body(dn_ref, qn_ref, dm_ref, qm_ref, g_ref,
              aW_ref, ab_ref, Wk_ref, bk_ref, Wq_ref, bq_ref, Wv_ref, bv_ref,
              Wa_ref, ba_ref, pri_ref, A_ref, M_ref, skip_ref,
              ln_g_ref, ln_b_ref, oW_ref, ob_ref,
              outd_ref, outq_ref):
    f32 = jnp.float32
    bf16 = jnp.bfloat16

    def mm(a, b):
        return jax.lax.dot_general(a, b, (((1,), (0,)), ((), ())),
                                   preferred_element_type=f32)

    def mm_nt(a, b):
        return jax.lax.dot_general(a, b, (((1,), (1,)), ((), ())),
                                   preferred_element_type=f32)

    def mm_tn(a, b):
        return jax.lax.dot_general(a, b, (((0,), (0,)), ((), ())),
                                   preferred_element_type=f32)

    def gelu_exact(x):
        return x * 0.5 * (1.0 + jax.lax.erf(x * np.float32(1.0 / np.sqrt(2.0))))

    eye_d = jnp.where(jax.lax.broadcasted_iota(jnp.int32, (D, D), 0)
                      == jax.lax.broadcasted_iota(jnp.int32, (D, D), 1),
                      0.0, 1.0).astype(f32)
    eye_q = jnp.where(jax.lax.broadcasted_iota(jnp.int32, (Q, Q), 0)
                      == jax.lax.broadcasted_iota(jnp.int32, (Q, Q), 1),
                      0.0, 1.0).astype(f32)
    z64 = jnp.zeros((DK, DK), f32)

    def blockdiag(m0, m1):
        return jnp.concatenate(
            [jnp.concatenate([m0, z64], axis=1),
             jnp.concatenate([z64, m1], axis=1)], axis=0)

    scale = np.float32(np.log2(np.e) / SQRT_DK)
    ablk = [blockdiag(A_ref[e, 0] * (pri_ref[e:e + 1, 0:1] * scale),
                      A_ref[e, 1] * (pri_ref[e:e + 1, 1:2] * scale))
            for e in range(8)]
    mblk = [blockdiag(M_ref[e, 0], M_ref[e, 1]) for e in range(8)]
    rels_of = {0: (0, 1, 4, 5), 1: (2, 3, 6, 7)}
    acat = {s: jnp.concatenate([ablk[e] for e in rels_of[s]], axis=1)
            for s in (0, 1)}
    mcat = {s: jnp.concatenate([mblk[e] for e in rels_of[s]], axis=1)
            for s in (0, 1)}
    Wkqv = [jnp.concatenate([Wk_ref[t], Wq_ref[t], Wv_ref[t]], axis=1)
            for t in (0, 1)]
    bkqv = [jnp.concatenate([bk_ref[t:t + 1, :], bq_ref[t:t + 1, :],
                             bv_ref[t:t + 1, :]], axis=1) for t in (0, 1)]
    ones_col = {0: jnp.full((D, 1), 1.0, f32), 1: jnp.full((Q, 1), 1.0, f32)}
    alpha = jax.nn.sigmoid(skip_ref[...])

    for bi in range(NB):
        dmf = dm_ref[bi].astype(f32)
        qmf = qm_ref[bi].astype(f32)
        g = g_ref[bi]

        h = [gelu_exact(mm(dn_ref[bi], aW_ref[0]) + ab_ref[0][None, :]),
             gelu_exact(mm(qn_ref[bi], aW_ref[1]) + ab_ref[1][None, :])]
        kqv = [mm(h[t], Wkqv[t]) + bkqv[t] for t in (0, 1)]
        kb = [kqv[t][:, 0:DM] for t in (0, 1)]
        qb = [kqv[t][:, DM:2 * DM] for t in (0, 1)]
        vb = [kqv[t][:, 2 * DM:3 * DM] for t in (0, 1)]

        pair = {
            (0, 0): mm_tn(dmf, dmf) * eye_d,
            (1, 1): mm_tn(qmf, qmf) * eye_q,
            (0, 1): mm_tn(dmf, qmf),
            (1, 0): mm_tn(qmf, dmf),
        }
        gf = g.astype(f32)
        gcf = 1.0 - gf
        gblk = {
            (0, 0): (gf[:D, :D], gcf[:D, :D]), (0, 1): (gf[:D, D:], gcf[:D, D:]),
            (1, 0): (gf[D:, :D], gcf[D:, :D]), (1, 1): (gf[D:, D:], gcf[D:, D:]),
        }

        sls = [slice(hh * DK, (hh + 1) * DK) for hh in range(H)]
        maskfs = {e: (pair[(s, t)] * gblk[(s, t)][0 if e < 4 else 1]).astype(bf16)
                  for (s, t, e) in _RELS}
        kwide = {s: mm(kb[s], acat[s]).astype(bf16) for s in (0, 1)}
        vwide = {s: mm(vb[s], mcat[s]).astype(bf16) for s in (0, 1)}
        k128s = {e: kwide[s][:, i * DM:(i + 1) * DM]
                 for s in (0, 1) for i, e in enumerate(rels_of[s])}
        v128s = {e: vwide[s][:, i * DM:(i + 1) * DM]
                 for s in (0, 1) for i, e in enumerate(rels_of[s])}
        qbh = [qb[t].astype(bf16) for t in (0, 1)]
        units = [(s, t, e, sl) for (s, t, e) in _RELS for sl in sls]
        ths = [mm_nt(k128s[e][:, sl], qbh[t][:, sl]) for (s, t, e, sl) in units]
        mxs = [jnp.max(th, axis=0, keepdims=True) for th in ths]
        ps = [jnp.exp2(th - mx).astype(bf16) * maskfs[e]
              for th, mx, (s, t, e, sl) in zip(ths, mxs, units)]
        vxs = [jnp.concatenate([v128s[e][:, sl], ones_col[s].astype(bf16)],
                               axis=1)
               for (s, t, e, sl) in units]
        rs = [mm_tn(vx, p) for vx, p in zip(vxs, ps)]
        heads = [r[:DK, :] / jnp.maximum(r[DK:DK + 1, :], 1e-30) for r in rs]
        aggT = [None, None]
        for i, (s, t, e, sl) in enumerate(units):
            if i % H == 0:
                contrib = jnp.concatenate(heads[i:i + H], axis=0)
                aggT[t] = contrib if aggT[t] is None else aggT[t] + contrib

        for t, out_ref in ((0, outd_ref), (1, outq_ref)):
            a_t = alpha[0:1, t:t + 1]
            trans = (mm_tn(aggT[t], Wa_ref[t]) * (0.25 * a_t)
                     + ba_ref[t][None, :] * a_t + h[t] * (1.0 - a_t))
            mu = jnp.mean(trans, axis=-1, keepdims=True)
            cent = trans - mu
            var = jnp.mean(cent * cent, axis=-1, keepdims=True)
            nh = (cent * jax.lax.rsqrt(var + EPS) * ln_g_ref[t][None, :]
                  + ln_b_ref[t][None, :])
            out_ref[bi] = mm(nh, oW_ref[...]) + ob_ref[0][None, :]


@jax.jit
def kernel(d_node, q_node, d_node_mask, q_node_mask, graph,
           adapt_W, adapt_b, Wk, bk, Wq, bq, Wv, bv, Wa, ba,
           rel_pri, rel_att, rel_msg, skip, ln_g, ln_b, out_W, out_b):
    f32 = jnp.float32
    dmask3 = d_node_mask.reshape(B, 1, D)
    qmask3 = q_node_mask.reshape(B, 1, Q)
    skip2 = skip.reshape(1, 2)
    ob2 = out_b.reshape(1, DM)

    def bspec(shape, batched):
        if batched:
            return pl.BlockSpec((NB,) + shape[1:],
                                lambda b: (b,) + (0,) * (len(shape) - 1))
        return pl.BlockSpec(shape, lambda b: (0,) * len(shape))

    in_specs = [
        bspec((B, D, DM), True),
        bspec((B, Q, DM), True),
        bspec((B, 1, D), True),
        bspec((B, 1, Q), True),
        bspec((B, D + Q, D + Q), True),
        bspec((2, DM, DM), False),
        bspec((2, DM), False),
        bspec((2, DM, DM), False),
        bspec((2, DM), False),
        bspec((2, DM, DM), False),
        bspec((2, DM), False),
        bspec((2, DM, DM), False),
        bspec((2, DM), False),
        bspec((2, DM, DM), False),
        bspec((2, DM), False),
        bspec((8, H), False),
        bspec((8, H, DK, DK), False),
        bspec((8, H, DK, DK), False),
        bspec((1, 2), False),
        bspec((2, DM), False),
        bspec((2, DM), False),
        bspec((DM, DM), False),
        bspec((1, DM), False),
    ]
    out_specs = [bspec((B, D, DM), True), bspec((B, Q, DM), True)]

    outd, outq = pl.pallas_call(
        _hgt_body,
        grid=(B // NB,),
        in_specs=in_specs,
        out_specs=out_specs,
        out_shape=[jax.ShapeDtypeStruct((B, D, DM), f32),
                   jax.ShapeDtypeStruct((B, Q, DM), f32)],
        compiler_params=pltpu.CompilerParams(
            dimension_semantics=("parallel",)),
    )(d_node, q_node, dmask3, qmask3, graph,
      adapt_W, adapt_b, Wk, bk, Wq, bq, Wv, bv, Wa, ba,
      rel_pri, rel_att, rel_msg, skip2, ln_g, ln_b, out_W, ob2)
    return outd, outq
